# baseline (speedup 1.0000x reference)
import functools

import jax
import jax.numpy as jnp
from jax.experimental import pallas as pl
from jax.experimental.pallas import tpu as pltpu

_SOBEL_EPS = 1e-6

_LANE_L1_FINE = 0
_LANE_SOB_FINE = 1
_LANE_L1_COARSE = 2
_LANE_SOB_COARSE = 3
_LANE_ERR = 4
_OUT_SUBLANES = 8
_OUT_LANES = 128


def _pick_block_images(bc, h, w):
    budget = 1024 * 1024
    cands = [d for d in range(1, bc + 1)
             if bc % d == 0 and ((d * h) % 8 == 0 or d == bc)]
    fitting = [d for d in cands if d * h * w * 4 <= budget]
    pool = fitting if fitting else [min(cands)]

    def score(d):
        nblk = bc // d
        return (nblk >= 2, nblk % 2 == 0, d)

    return max(pool, key=score)


def _roll(x, shift, axis, fast):
    if fast:
        return pltpu.roll(x, shift % x.shape[axis], axis)
    return jnp.roll(x, shift, axis)


def _border_masks(rows, cols, img_h):
    row = jax.lax.broadcasted_iota(jnp.int32, (rows, cols), 0)
    col = jax.lax.broadcasted_iota(jnp.int32, (rows, cols), 1)
    if rows == img_h:
        r = row
    elif (img_h & (img_h - 1)) == 0:
        r = jnp.bitwise_and(row, img_h - 1)
    else:
        r = row % img_h
    return r == 0, r == img_h - 1, col == 0, col == cols - 1


def _sobel_mag8(x, top, bot, left, right, fr, fc):
    x_u = jnp.where(top, x, _roll(x, 1, 0, fr))
    x_d = jnp.where(bot, x, _roll(x, -1, 0, fr))
    x_l = jnp.where(left, x, _roll(x, 1, 1, fc))
    x_r = jnp.where(right, x, _roll(x, -1, 1, fc))
    v = x_u + 2.0 * x + x_d
    h = x_l + 2.0 * x + x_r
    v_l = jnp.where(left, v, _roll(v, 1, 1, fc))
    v_r = jnp.where(right, v, _roll(v, -1, 1, fc))
    h_u = jnp.where(top, h, _roll(h, 1, 0, fr))
    h_d = jnp.where(bot, h, _roll(h, -1, 0, fr))
    gx = v_r - v_l
    gy = h_d - h_u
    return jnp.sqrt(gx * gx + gy * gy + 64.0 * _SOBEL_EPS)


def _loss_body(p_ref, t_ref, c_ref, e_ref, mh_ref, mw_ref, out_ref,
               *, img_h, n_img):
    rows, cols = p_ref.shape
    hh = c_ref.shape[0] // n_img
    fr = rows % 8 == 0
    fc = cols % 128 == 0

    p = p_ref[...]
    t = t_ref[...]
    mh = mh_ref[...]
    mw = mw_ref[...]

    def upsample(ref):
        xb = ref[...].astype(jnp.bfloat16)
        y = jnp.dot(xb, mw, preferred_element_type=jnp.float32)
        yb = y.astype(jnp.bfloat16)
        zs = [jnp.dot(mh, yb[i * hh:(i + 1) * hh],
                      preferred_element_type=jnp.float32)
              for i in range(n_img)]
        return zs[0] if n_img == 1 else jnp.concatenate(zs, axis=0)

    c = upsample(c_ref)
    e = upsample(e_ref)

    top, bot, left, right = _border_masks(rows, cols, img_h)
    mag_t = _sobel_mag8(t, top, bot, left, right, fr, fc)
    mag_p = _sobel_mag8(p, top, bot, left, right, fr, fc)
    mag_c = _sobel_mag8(c, top, bot, left, right, fr, fc)

    entries = [
        (_LANE_L1_FINE, jnp.sum(jnp.abs(p - t))),
        (_LANE_SOB_FINE, 0.125 * jnp.sum(jnp.abs(mag_p - mag_t))),
        (_LANE_L1_COARSE, jnp.sum(jnp.abs(c - t))),
        (_LANE_SOB_COARSE, 0.125 * jnp.sum(jnp.abs(mag_c - mag_t))),
        (_LANE_ERR, jnp.sum(jnp.abs(e - jnp.abs(c - t)))),
    ]
    lane = jax.lax.broadcasted_iota(
        jnp.int32, (_OUT_SUBLANES, _OUT_LANES), 1)
    acc = jnp.zeros((_OUT_SUBLANES, _OUT_LANES), jnp.float32)
    for idx, s in entries:
        acc = acc + jnp.where(lane == idx, s, 0.0)
    out_ref[...] = acc.reshape(out_ref.shape)


def _upsample_operators(hh, ww, H, W):
    mh = jax.image.resize(jnp.eye(hh, dtype=jnp.float32), (H, hh),
                          method='bilinear').astype(jnp.bfloat16)
    mw = jax.image.resize(jnp.eye(ww, dtype=jnp.float32), (ww, W),
                          method='bilinear').astype(jnp.bfloat16)
    return mh, mw


def kernel(pred_pha_fine, true_pha, pred_pha_corse, pred_err):
    B, C, H, W = true_pha.shape
    bc = B * C
    hh, ww = pred_pha_corse.shape[2:]
    tbc = _pick_block_images(bc, H, W)
    nblk = bc // tbc
    rows = tbc * H

    p2 = pred_pha_fine.reshape(bc * H, W)
    t2 = true_pha.reshape(bc * H, W)
    c2 = pred_pha_corse.reshape(bc * hh, ww)
    e2 = pred_err.reshape(bc * hh, ww)
    mh, mw = _upsample_operators(hh, ww, H, W)

    full_spec = pl.BlockSpec((rows, W), lambda i: (i, 0))
    half_spec = pl.BlockSpec((tbc * hh, ww), lambda i: (i, 0))
    fixed = lambda shape: pl.BlockSpec(shape, lambda i: (0, 0))

    out_spec = pl.BlockSpec((1, _OUT_SUBLANES, _OUT_LANES), lambda i: (i, 0, 0))
    out_shape = jax.ShapeDtypeStruct(
        (nblk, _OUT_SUBLANES, _OUT_LANES), jnp.float32)

    body = functools.partial(_loss_body, img_h=H, n_img=tbc)
    sums = pl.pallas_call(
        body,
        out_shape=out_shape,
        grid=(nblk,),
        in_specs=[full_spec, full_spec, half_spec, half_spec,
                  fixed(mh.shape), fixed(mw.shape)],
        out_specs=out_spec,
        compiler_params=pltpu.CompilerParams(
            dimension_semantics=("parallel",),
            vmem_limit_bytes=96 * 1024 * 1024),
    )(p2, t2, c2, e2, mh, mw)

    n = jnp.float32(bc * H * W)
    return {
        'main_loss': (jnp.sum(sums[:, 0, _LANE_L1_FINE]) +
                      jnp.sum(sums[:, 0, _LANE_SOB_FINE])) / n,
        'coarse_pred_loss': (jnp.sum(sums[:, 0, _LANE_L1_COARSE]) +
                             jnp.sum(sums[:, 0, _LANE_SOB_COARSE])) / n,
        'pred_err_loss': jnp.sum(sums[:, 0, _LANE_ERR]) / n,
    }


# speedup vs baseline: 1.1331x; 1.1331x over previous
import jax
import jax.numpy as jnp
from jax.experimental import pallas as pl
from jax.experimental.pallas import tpu as pltpu

_SOBEL_EPS = 1e-6

_LANE_L1_FINE = 0
_LANE_SOB_FINE = 1
_LANE_L1_COARSE = 2
_LANE_SOB_COARSE = 3
_LANE_ERR = 4
_OUT_SUBLANES = 8
_OUT_LANES = 128


def _shift_down(x):
    return jnp.concatenate([x[:1], x[:-1]], axis=0)


def _shift_up(x):
    return jnp.concatenate([x[1:], x[-1:]], axis=0)


def _shift_right(x):
    return jnp.concatenate([x[:, :1], x[:, :-1]], axis=1)


def _shift_left(x):
    return jnp.concatenate([x[:, 1:], x[:, -1:]], axis=1)


def _sobel_mag8(x):
    x_l = _shift_right(x)
    x_r = _shift_left(x)
    d = x_r - x_l
    s = x_l + 2.0 * x + x_r
    gx = _shift_down(d) + 2.0 * d + _shift_up(d)
    gy = _shift_up(s) - _shift_down(s)
    return jnp.sqrt(gx * gx + gy * gy + 64.0 * _SOBEL_EPS)


def _loss_body(p_ref, t_ref, c_ref, e_ref, mh_ref, mw_ref, out_ref):
    p = p_ref[...]
    t = t_ref[...]
    mh = mh_ref[...]
    mw = mw_ref[...]

    def upsample(ref):
        xb = ref[...].astype(jnp.bfloat16)
        y = jnp.dot(xb, mw, preferred_element_type=jnp.float32)
        return jnp.dot(mh, y.astype(jnp.bfloat16),
                       preferred_element_type=jnp.float32)

    c = upsample(c_ref)
    e = upsample(e_ref)

    mag_t = _sobel_mag8(t)
    mag_p = _sobel_mag8(p)
    mag_c = _sobel_mag8(c)

    ct = jnp.abs(c - t)
    entries = [
        (_LANE_L1_FINE, jnp.sum(jnp.abs(p - t))),
        (_LANE_SOB_FINE, 0.125 * jnp.sum(jnp.abs(mag_p - mag_t))),
        (_LANE_L1_COARSE, jnp.sum(ct)),
        (_LANE_SOB_COARSE, 0.125 * jnp.sum(jnp.abs(mag_c - mag_t))),
        (_LANE_ERR, jnp.sum(jnp.abs(e - ct))),
    ]
    lane = jax.lax.broadcasted_iota(
        jnp.int32, (_OUT_SUBLANES, _OUT_LANES), 1)
    acc = jnp.zeros((_OUT_SUBLANES, _OUT_LANES), jnp.float32)
    for idx, s in entries:
        acc = acc + jnp.where(lane == idx, s, 0.0)
    out_ref[...] = acc.reshape(out_ref.shape)


def _upsample_operators(hh, ww, H, W):
    mh = jax.image.resize(jnp.eye(hh, dtype=jnp.float32), (H, hh),
                          method='bilinear').astype(jnp.bfloat16)
    mw = jax.image.resize(jnp.eye(ww, dtype=jnp.float32), (ww, W),
                          method='bilinear').astype(jnp.bfloat16)
    return mh, mw


def kernel(pred_pha_fine, true_pha, pred_pha_corse, pred_err):
    B, C, H, W = true_pha.shape
    bc = B * C
    hh, ww = pred_pha_corse.shape[2:]
    nblk = bc

    p2 = pred_pha_fine.reshape(bc * H, W)
    t2 = true_pha.reshape(bc * H, W)
    c2 = pred_pha_corse.reshape(bc * hh, ww)
    e2 = pred_err.reshape(bc * hh, ww)
    mh, mw = _upsample_operators(hh, ww, H, W)

    full_spec = pl.BlockSpec((H, W), lambda i: (i, 0))
    half_spec = pl.BlockSpec((hh, ww), lambda i: (i, 0))
    fixed = lambda shape: pl.BlockSpec(shape, lambda i: (0, 0))

    out_spec = pl.BlockSpec((1, _OUT_SUBLANES, _OUT_LANES), lambda i: (i, 0, 0))
    out_shape = jax.ShapeDtypeStruct(
        (nblk, _OUT_SUBLANES, _OUT_LANES), jnp.float32)

    sums = pl.pallas_call(
        _loss_body,
        out_shape=out_shape,
        grid=(nblk,),
        in_specs=[full_spec, full_spec, half_spec, half_spec,
                  fixed(mh.shape), fixed(mw.shape)],
        out_specs=out_spec,
        compiler_params=pltpu.CompilerParams(
            dimension_semantics=("parallel",),
            vmem_limit_bytes=96 * 1024 * 1024),
    )(p2, t2, c2, e2, mh, mw)

    n = jnp.float32(bc * H * W)
    return {
        'main_loss': (jnp.sum(sums[:, 0, _LANE_L1_FINE]) +
                      jnp.sum(sums[:, 0, _LANE_SOB_FINE])) / n,
        'coarse_pred_loss': (jnp.sum(sums[:, 0, _LANE_L1_COARSE]) +
                             jnp.sum(sums[:, 0, _LANE_SOB_COARSE])) / n,
        'pred_err_loss': jnp.sum(sums[:, 0, _LANE_ERR]) / n,
    }


# speedup vs baseline: 1.4441x; 1.2745x over previous
import jax
import jax.numpy as jnp
from jax.experimental import pallas as pl
from jax.experimental.pallas import tpu as pltpu

_SOBEL_EPS = 1e-6

_LANE_L1_FINE = 0
_LANE_SOB_FINE = 1
_LANE_L1_COARSE = 2
_LANE_SOB_COARSE = 3
_LANE_ERR = 4
_OUT_SUBLANES = 8
_OUT_LANES = 128


def _shift_down(x):
    return jnp.concatenate([x[:1], x[:-1]], axis=0)


def _shift_up(x):
    return jnp.concatenate([x[1:], x[-1:]], axis=0)


def _shift_right(x):
    return jnp.concatenate([x[:, :1], x[:, :-1]], axis=1)


def _shift_left(x):
    return jnp.concatenate([x[:, 1:], x[:, -1:]], axis=1)


def _sobel_mag8(x):
    x_l = _shift_right(x)
    x_r = _shift_left(x)
    d = x_r - x_l
    s = x_l + 2.0 * x + x_r
    gx = _shift_down(d) + 2.0 * d + _shift_up(d)
    gy = _shift_up(s) - _shift_down(s)
    g2 = gx * gx + gy * gy + 64.0 * _SOBEL_EPS
    return g2 * jax.lax.rsqrt(g2)


def _loss_body(p_ref, t_ref, c_ref, e_ref, mh_ref, mw_ref, out_ref):
    p = p_ref[...].astype(jnp.bfloat16)
    t = t_ref[...].astype(jnp.bfloat16)
    mh = mh_ref[...]
    mw = mw_ref[...]

    def upsample(ref):
        xb = ref[...].astype(jnp.bfloat16)
        y = jnp.dot(xb, mw, preferred_element_type=jnp.float32)
        return jnp.dot(mh, y.astype(jnp.bfloat16),
                       preferred_element_type=jnp.float32).astype(jnp.bfloat16)

    c = upsample(c_ref)
    e = upsample(e_ref)

    mag_t = _sobel_mag8(t)
    mag_p = _sobel_mag8(p)
    mag_c = _sobel_mag8(c)

    ct = jnp.abs(c - t)
    fsum = lambda x: jnp.sum(x, dtype=jnp.float32)
    entries = [
        (_LANE_L1_FINE, fsum(jnp.abs(p - t))),
        (_LANE_SOB_FINE, 0.125 * fsum(jnp.abs(mag_p - mag_t))),
        (_LANE_L1_COARSE, fsum(ct)),
        (_LANE_SOB_COARSE, 0.125 * fsum(jnp.abs(mag_c - mag_t))),
        (_LANE_ERR, fsum(jnp.abs(e - ct))),
    ]
    lane = jax.lax.broadcasted_iota(
        jnp.int32, (_OUT_SUBLANES, _OUT_LANES), 1)
    acc = jnp.zeros((_OUT_SUBLANES, _OUT_LANES), jnp.float32)
    for idx, s in entries:
        acc = acc + jnp.where(lane == idx, s, 0.0)
    out_ref[...] = acc.reshape(out_ref.shape)


def _upsample_operators(hh, ww, H, W):
    mh = jax.image.resize(jnp.eye(hh, dtype=jnp.float32), (H, hh),
                          method='bilinear').astype(jnp.bfloat16)
    mw = jax.image.resize(jnp.eye(ww, dtype=jnp.float32), (ww, W),
                          method='bilinear').astype(jnp.bfloat16)
    return mh, mw


def kernel(pred_pha_fine, true_pha, pred_pha_corse, pred_err):
    B, C, H, W = true_pha.shape
    bc = B * C
    hh, ww = pred_pha_corse.shape[2:]
    nblk = bc

    p2 = pred_pha_fine.reshape(bc * H, W)
    t2 = true_pha.reshape(bc * H, W)
    c2 = pred_pha_corse.reshape(bc * hh, ww)
    e2 = pred_err.reshape(bc * hh, ww)
    mh, mw = _upsample_operators(hh, ww, H, W)

    full_spec = pl.BlockSpec((H, W), lambda i: (i, 0))
    half_spec = pl.BlockSpec((hh, ww), lambda i: (i, 0))
    fixed = lambda shape: pl.BlockSpec(shape, lambda i: (0, 0))

    out_spec = pl.BlockSpec((1, _OUT_SUBLANES, _OUT_LANES), lambda i: (i, 0, 0))
    out_shape = jax.ShapeDtypeStruct(
        (nblk, _OUT_SUBLANES, _OUT_LANES), jnp.float32)

    sums = pl.pallas_call(
        _loss_body,
        out_shape=out_shape,
        grid=(nblk,),
        in_specs=[full_spec, full_spec, half_spec, half_spec,
                  fixed(mh.shape), fixed(mw.shape)],
        out_specs=out_spec,
        compiler_params=pltpu.CompilerParams(
            dimension_semantics=("parallel",),
            vmem_limit_bytes=96 * 1024 * 1024),
    )(p2, t2, c2, e2, mh, mw)

    n = jnp.float32(bc * H * W)
    return {
        'main_loss': (jnp.sum(sums[:, 0, _LANE_L1_FINE]) +
                      jnp.sum(sums[:, 0, _LANE_SOB_FINE])) / n,
        'coarse_pred_loss': (jnp.sum(sums[:, 0, _LANE_L1_COARSE]) +
                             jnp.sum(sums[:, 0, _LANE_SOB_COARSE])) / n,
        'pred_err_loss': jnp.sum(sums[:, 0, _LANE_ERR]) / n,
    }


# speedup vs baseline: 1.8699x; 1.2948x over previous
import jax
import jax.numpy as jnp
from jax.experimental import pallas as pl
from jax.experimental.pallas import tpu as pltpu

_SOBEL_EPS = 1e-6

_LANE_L1_FINE = 0
_LANE_SOB_FINE = 1
_LANE_L1_COARSE = 2
_LANE_SOB_COARSE = 3
_LANE_ERR = 4
_OUT_SUBLANES = 8
_OUT_LANES = 128


def _shift_down(x):
    return jnp.concatenate([x[:1], x[:-1]], axis=0)


def _shift_up(x):
    return jnp.concatenate([x[1:], x[-1:]], axis=0)


def _shift_right(x):
    return jnp.concatenate([x[:, :1], x[:, :-1]], axis=1)


def _shift_left(x):
    return jnp.concatenate([x[:, 1:], x[:, -1:]], axis=1)


def _sobel_mag8(x):
    x_l = _shift_right(x)
    x_r = _shift_left(x)
    d = x_r - x_l
    s = x_l + 2.0 * x + x_r
    gx = _shift_down(d) + 2.0 * d + _shift_up(d)
    gy = _shift_up(s) - _shift_down(s)
    g2 = gx * gx + gy * gy + 64.0 * _SOBEL_EPS
    return g2 * jax.lax.rsqrt(g2)


def _loss_body(p_ref, t_ref, c_ref, e_ref, mh_ref, mw_ref, out_ref,
               *, img_h, low_h, n_img):
    mh = mh_ref[...]
    mw = mw_ref[...]

    def upsample(ref, i):
        xb = ref[i * low_h:(i + 1) * low_h].astype(jnp.bfloat16)
        y = jnp.dot(xb, mw, preferred_element_type=jnp.float32)
        return jnp.dot(mh, y.astype(jnp.bfloat16),
                       preferred_element_type=jnp.float32).astype(jnp.bfloat16)

    sums = [jnp.float32(0.0)] * 5
    fsum = lambda x: jnp.sum(x, dtype=jnp.float32)
    for i in range(n_img):
        sl = slice(i * img_h, (i + 1) * img_h)
        p = p_ref[sl].astype(jnp.bfloat16)
        t = t_ref[sl].astype(jnp.bfloat16)
        c = upsample(c_ref, i)
        e = upsample(e_ref, i)

        mag_t = _sobel_mag8(t)
        mag_p = _sobel_mag8(p)
        mag_c = _sobel_mag8(c)

        ct = jnp.abs(c - t)
        sums[0] += fsum(jnp.abs(p - t))
        sums[1] += 0.125 * fsum(jnp.abs(mag_p - mag_t))
        sums[2] += fsum(ct)
        sums[3] += 0.125 * fsum(jnp.abs(mag_c - mag_t))
        sums[4] += fsum(jnp.abs(e - ct))

    entries = [
        (_LANE_L1_FINE, sums[0]),
        (_LANE_SOB_FINE, sums[1]),
        (_LANE_L1_COARSE, sums[2]),
        (_LANE_SOB_COARSE, sums[3]),
        (_LANE_ERR, sums[4]),
    ]
    lane = jax.lax.broadcasted_iota(
        jnp.int32, (_OUT_SUBLANES, _OUT_LANES), 1)
    acc = jnp.zeros((_OUT_SUBLANES, _OUT_LANES), jnp.float32)
    for idx, s in entries:
        acc = acc + jnp.where(lane == idx, s, 0.0)
    out_ref[...] = acc.reshape(out_ref.shape)


def _upsample_operators(hh, ww, H, W):
    mh = jax.image.resize(jnp.eye(hh, dtype=jnp.float32), (H, hh),
                          method='bilinear').astype(jnp.bfloat16)
    mw = jax.image.resize(jnp.eye(ww, dtype=jnp.float32), (ww, W),
                          method='bilinear').astype(jnp.bfloat16)
    return mh, mw


def kernel(pred_pha_fine, true_pha, pred_pha_corse, pred_err):
    B, C, H, W = true_pha.shape
    bc = B * C
    hh, ww = pred_pha_corse.shape[2:]
    tbc = 4 if bc % 4 == 0 else (2 if bc % 2 == 0 else 1)
    nblk = bc // tbc

    p2 = pred_pha_fine.reshape(bc * H, W)
    t2 = true_pha.reshape(bc * H, W)
    c2 = pred_pha_corse.reshape(bc * hh, ww)
    e2 = pred_err.reshape(bc * hh, ww)
    mh, mw = _upsample_operators(hh, ww, H, W)

    full_spec = pl.BlockSpec((tbc * H, W), lambda i: (i, 0))
    half_spec = pl.BlockSpec((tbc * hh, ww), lambda i: (i, 0))
    fixed = lambda shape: pl.BlockSpec(shape, lambda i: (0, 0))

    out_spec = pl.BlockSpec((1, _OUT_SUBLANES, _OUT_LANES), lambda i: (i, 0, 0))
    out_shape = jax.ShapeDtypeStruct(
        (nblk, _OUT_SUBLANES, _OUT_LANES), jnp.float32)

    import functools as _ft
    body = _ft.partial(_loss_body, img_h=H, low_h=hh, n_img=tbc)
    sums = pl.pallas_call(
        body,
        out_shape=out_shape,
        grid=(nblk,),
        in_specs=[full_spec, full_spec, half_spec, half_spec,
                  fixed(mh.shape), fixed(mw.shape)],
        out_specs=out_spec,
        compiler_params=pltpu.CompilerParams(
            dimension_semantics=("parallel",),
            vmem_limit_bytes=96 * 1024 * 1024),
    )(p2, t2, c2, e2, mh, mw)

    n = jnp.float32(bc * H * W)
    return {
        'main_loss': (jnp.sum(sums[:, 0, _LANE_L1_FINE]) +
                      jnp.sum(sums[:, 0, _LANE_SOB_FINE])) / n,
        'coarse_pred_loss': (jnp.sum(sums[:, 0, _LANE_L1_COARSE]) +
                             jnp.sum(sums[:, 0, _LANE_SOB_COARSE])) / n,
        'pred_err_loss': jnp.sum(sums[:, 0, _LANE_ERR]) / n,
    }


# speedup vs baseline: 1.9356x; 1.0351x over previous
import jax
import jax.numpy as jnp
from jax.experimental import pallas as pl
from jax.experimental.pallas import tpu as pltpu

_SOBEL_EPS = 1e-6

_LANE_L1_FINE = 0
_LANE_SOB_FINE = 1
_LANE_L1_COARSE = 2
_LANE_SOB_COARSE = 3
_LANE_ERR = 4
_OUT_SUBLANES = 8
_OUT_LANES = 128


def _shift_down(x):
    return jnp.concatenate([x[:1], x[:-1]], axis=0)


def _shift_up(x):
    return jnp.concatenate([x[1:], x[-1:]], axis=0)


def _shift_right(x):
    return jnp.concatenate([x[:, :1], x[:, :-1]], axis=1)


def _shift_left(x):
    return jnp.concatenate([x[:, 1:], x[:, -1:]], axis=1)


def _sobel_mag8(x):
    x_u = _shift_down(x)
    x_d = _shift_up(x)
    sv = x_u + 2.0 * x + x_d
    dv = x_d - x_u
    gx = _shift_left(sv) - _shift_right(sv)
    gy = _shift_right(dv) + 2.0 * dv + _shift_left(dv)
    g2 = gx * gx + gy * gy + 64.0 * _SOBEL_EPS
    return g2 * jax.lax.rsqrt(g2)


def _loss_body(p_ref, t_ref, c_ref, e_ref, mh_ref, mw_ref, out_ref,
               *, img_h, low_h, n_img):
    mh = mh_ref[...]
    mw = mw_ref[...]

    def upsample(ref, i):
        xb = ref[i * low_h:(i + 1) * low_h].astype(jnp.bfloat16)
        y = jnp.dot(xb, mw, preferred_element_type=jnp.float32)
        return jnp.dot(mh, y.astype(jnp.bfloat16),
                       preferred_element_type=jnp.float32).astype(jnp.bfloat16)

    sums = [jnp.float32(0.0)] * 5
    fsum = lambda x: jnp.sum(x, dtype=jnp.float32)
    for i in range(n_img):
        sl = slice(i * img_h, (i + 1) * img_h)
        p = p_ref[sl].astype(jnp.bfloat16)
        t = t_ref[sl].astype(jnp.bfloat16)
        c = upsample(c_ref, i)
        e = upsample(e_ref, i)

        mag_t = _sobel_mag8(t)
        mag_p = _sobel_mag8(p)
        mag_c = _sobel_mag8(c)

        ct = jnp.abs(c - t)
        sums[0] += fsum(jnp.abs(p - t))
        sums[1] += 0.125 * fsum(jnp.abs(mag_p - mag_t))
        sums[2] += fsum(ct)
        sums[3] += 0.125 * fsum(jnp.abs(mag_c - mag_t))
        sums[4] += fsum(jnp.abs(e - ct))

    entries = [
        (_LANE_L1_FINE, sums[0]),
        (_LANE_SOB_FINE, sums[1]),
        (_LANE_L1_COARSE, sums[2]),
        (_LANE_SOB_COARSE, sums[3]),
        (_LANE_ERR, sums[4]),
    ]
    lane = jax.lax.broadcasted_iota(
        jnp.int32, (_OUT_SUBLANES, _OUT_LANES), 1)
    acc = jnp.zeros((_OUT_SUBLANES, _OUT_LANES), jnp.float32)
    for idx, s in entries:
        acc = acc + jnp.where(lane == idx, s, 0.0)
    out_ref[...] = acc.reshape(out_ref.shape)


def _upsample_operators(hh, ww, H, W):
    mh = jax.image.resize(jnp.eye(hh, dtype=jnp.float32), (H, hh),
                          method='bilinear').astype(jnp.bfloat16)
    mw = jax.image.resize(jnp.eye(ww, dtype=jnp.float32), (ww, W),
                          method='bilinear').astype(jnp.bfloat16)
    return mh, mw


def kernel(pred_pha_fine, true_pha, pred_pha_corse, pred_err):
    B, C, H, W = true_pha.shape
    bc = B * C
    hh, ww = pred_pha_corse.shape[2:]
    tbc = 8 if bc % 8 == 0 else (4 if bc % 4 == 0 else (2 if bc % 2 == 0 else 1))
    nblk = bc // tbc

    p2 = pred_pha_fine.reshape(bc * H, W)
    t2 = true_pha.reshape(bc * H, W)
    c2 = pred_pha_corse.reshape(bc * hh, ww)
    e2 = pred_err.reshape(bc * hh, ww)
    mh, mw = _upsample_operators(hh, ww, H, W)

    full_spec = pl.BlockSpec((tbc * H, W), lambda i: (i, 0))
    half_spec = pl.BlockSpec((tbc * hh, ww), lambda i: (i, 0))
    fixed = lambda shape: pl.BlockSpec(shape, lambda i: (0, 0))

    out_spec = pl.BlockSpec((1, _OUT_SUBLANES, _OUT_LANES), lambda i: (i, 0, 0))
    out_shape = jax.ShapeDtypeStruct(
        (nblk, _OUT_SUBLANES, _OUT_LANES), jnp.float32)

    import functools as _ft
    body = _ft.partial(_loss_body, img_h=H, low_h=hh, n_img=tbc)
    sums = pl.pallas_call(
        body,
        out_shape=out_shape,
        grid=(nblk,),
        in_specs=[full_spec, full_spec, half_spec, half_spec,
                  fixed(mh.shape), fixed(mw.shape)],
        out_specs=out_spec,
        compiler_params=pltpu.CompilerParams(
            dimension_semantics=("parallel",),
            vmem_limit_bytes=96 * 1024 * 1024),
    )(p2, t2, c2, e2, mh, mw)

    n = jnp.float32(bc * H * W)
    return {
        'main_loss': (jnp.sum(sums[:, 0, _LANE_L1_FINE]) +
                      jnp.sum(sums[:, 0, _LANE_SOB_FINE])) / n,
        'coarse_pred_loss': (jnp.sum(sums[:, 0, _LANE_L1_COARSE]) +
                             jnp.sum(sums[:, 0, _LANE_SOB_COARSE])) / n,
        'pred_err_loss': jnp.sum(sums[:, 0, _LANE_ERR]) / n,
    }


# speedup vs baseline: 1.9625x; 1.0139x over previous
import functools

import numpy as np

import jax
import jax.numpy as jnp
from jax.experimental import pallas as pl
from jax.experimental.pallas import tpu as pltpu

_SOBEL_EPS = 1e-6

_LANE_L1_FINE = 0
_LANE_SOB_FINE = 1
_LANE_L1_COARSE = 2
_LANE_SOB_COARSE = 3
_LANE_ERR = 4
_OUT_SUBLANES = 8
_OUT_LANES = 128


def _shift_down(x):
    return jnp.concatenate([x[:1], x[:-1]], axis=0)


def _shift_up(x):
    return jnp.concatenate([x[1:], x[-1:]], axis=0)


def _shift_right(x):
    return jnp.concatenate([x[:, :1], x[:, :-1]], axis=1)


def _shift_left(x):
    return jnp.concatenate([x[:, 1:], x[:, -1:]], axis=1)


def _sobel_mag8(x):
    x_u = _shift_down(x)
    x_d = _shift_up(x)
    sv = x_u + 2.0 * x + x_d
    dv = x_d - x_u
    gx = _shift_left(sv) - _shift_right(sv)
    gy = _shift_right(dv) + 2.0 * dv + _shift_left(dv)
    g2 = gx * gx + gy * gy + 64.0 * _SOBEL_EPS
    return g2 * jax.lax.rsqrt(g2)


def _loss_body(p_ref, t_ref, c_ref, e_ref, mh_ref, mw_ref, out_ref,
               *, img_h, low_h, n_img):
    mh = mh_ref[...]
    mw = mw_ref[...]

    def upsample(ref, i):
        xb = ref[i * low_h:(i + 1) * low_h].astype(jnp.bfloat16)
        y = jnp.dot(xb, mw, preferred_element_type=jnp.float32)
        return jnp.dot(mh, y.astype(jnp.bfloat16),
                       preferred_element_type=jnp.float32).astype(jnp.bfloat16)

    sums = [jnp.float32(0.0)] * 5
    fsum = lambda x: jnp.sum(x, dtype=jnp.float32)
    for i in range(n_img):
        sl = slice(i * img_h, (i + 1) * img_h)
        p = p_ref[sl].astype(jnp.bfloat16)
        t = t_ref[sl].astype(jnp.bfloat16)
        c = upsample(c_ref, i)
        e = upsample(e_ref, i)

        mag_t = _sobel_mag8(t)
        mag_p = _sobel_mag8(p)
        mag_c = _sobel_mag8(c)

        ct = jnp.abs(c - t)
        sums[0] += fsum(jnp.abs(p - t))
        sums[1] += 0.125 * fsum(jnp.abs(mag_p - mag_t))
        sums[2] += fsum(ct)
        sums[3] += 0.125 * fsum(jnp.abs(mag_c - mag_t))
        sums[4] += fsum(jnp.abs(e - ct))

    lane = jax.lax.broadcasted_iota(
        jnp.int32, (_OUT_SUBLANES, _OUT_LANES), 1)
    acc = jnp.zeros((_OUT_SUBLANES, _OUT_LANES), jnp.float32)
    for idx, s in zip((_LANE_L1_FINE, _LANE_SOB_FINE, _LANE_L1_COARSE,
                       _LANE_SOB_COARSE, _LANE_ERR), sums):
        acc = acc + jnp.where(lane == idx, s, 0.0)

    @pl.when(pl.program_id(0) == 0)
    def _init():
        out_ref[...] = acc

    @pl.when(pl.program_id(0) != 0)
    def _accum():
        out_ref[...] += acc


def _bilinear_operator(n_src, n_dst):
    i = np.arange(n_dst, dtype=np.float64)
    src = (i + 0.5) * (n_src / n_dst) - 0.5
    lo = np.floor(src).astype(np.int64)
    w = src - lo
    hi = np.clip(lo + 1, 0, n_src - 1)
    lo = np.clip(lo, 0, n_src - 1)
    m = np.zeros((n_dst, n_src), dtype=np.float32)
    m[i.astype(np.int64), lo] += (1.0 - w).astype(np.float32)
    m[i.astype(np.int64), hi] += w.astype(np.float32)
    return m


def kernel(pred_pha_fine, true_pha, pred_pha_corse, pred_err):
    B, C, H, W = true_pha.shape
    bc = B * C
    hh, ww = pred_pha_corse.shape[2:]
    tbc = 8 if bc % 8 == 0 else (4 if bc % 4 == 0 else (2 if bc % 2 == 0 else 1))
    nblk = bc // tbc

    p2 = pred_pha_fine.reshape(bc * H, W)
    t2 = true_pha.reshape(bc * H, W)
    c2 = pred_pha_corse.reshape(bc * hh, ww)
    e2 = pred_err.reshape(bc * hh, ww)
    mh = jnp.asarray(_bilinear_operator(hh, H), dtype=jnp.bfloat16)
    mw = jnp.asarray(_bilinear_operator(ww, W).T, dtype=jnp.bfloat16)

    full_spec = pl.BlockSpec((tbc * H, W), lambda i: (i, 0))
    half_spec = pl.BlockSpec((tbc * hh, ww), lambda i: (i, 0))
    fixed = lambda shape: pl.BlockSpec(shape, lambda i: (0, 0))

    out_spec = pl.BlockSpec((_OUT_SUBLANES, _OUT_LANES), lambda i: (0, 0))
    out_shape = jax.ShapeDtypeStruct((_OUT_SUBLANES, _OUT_LANES), jnp.float32)

    body = functools.partial(_loss_body, img_h=H, low_h=hh, n_img=tbc)
    sums = pl.pallas_call(
        body,
        out_shape=out_shape,
        grid=(nblk,),
        in_specs=[full_spec, full_spec, half_spec, half_spec,
                  fixed(mh.shape), fixed(mw.shape)],
        out_specs=out_spec,
        compiler_params=pltpu.CompilerParams(
            dimension_semantics=("arbitrary",),
            vmem_limit_bytes=96 * 1024 * 1024),
    )(p2, t2, c2, e2, mh, mw)

    n = jnp.float32(bc * H * W)
    row = sums[0]
    return {
        'main_loss': (row[_LANE_L1_FINE] + row[_LANE_SOB_FINE]) / n,
        'coarse_pred_loss': (row[_LANE_L1_COARSE] + row[_LANE_SOB_COARSE]) / n,
        'pred_err_loss': row[_LANE_ERR] / n,
    }
